# baseline (speedup 1.0000x reference)
import functools

import jax
import jax.numpy as jnp
from jax import lax
from jax.experimental import pallas as pl
from jax.experimental.pallas import tpu as pltpu
from jax.experimental.pallas import tpu_sc as plsc

B = 4096
K = 10
L = 16
NC = 2
NS = 16
NW = NC * NS
SC_ROWS = 1024
TC_ROWS = B - SC_ROWS
RBLK = 256
ROWS_PER_W = SC_ROWS // NW
RCH = 8
NCH = ROWS_PER_W // RCH
CHUNKS = B // L
NEG = -3.0e38


def _max_splat(x, lane):
    for k in (1, 2, 4, 8):
        x = jnp.maximum(x, x.at[lane ^ k].get(mode="promise_in_bounds"))
    return x


SIZES = (K, 5, 3, 2)
GRPS = CHUNKS // 4


def _ins_sorted(regs, v):
    n = len(regs)
    for i in range(n):
        if i < n - 1:
            lo = jnp.minimum(regs[i], v)
            regs[i] = jnp.maximum(regs[i], v)
            v = lo
        else:
            regs[i] = jnp.maximum(regs[i], v)


def _row_candidates(buf, j):
    init = tuple(jnp.full((L,), NEG, jnp.float32) for _ in range(sum(SIZES)))

    def grp(g, carry):
        regs = list(carry)
        vs = [buf[j, pl.ds((4 * g + i) * L, L)] for i in range(4)]
        for a, b in ((0, 1), (2, 3), (0, 2), (1, 3), (1, 2)):
            hi = jnp.maximum(vs[a], vs[b])
            vs[b] = jnp.minimum(vs[a], vs[b])
            vs[a] = hi
        off = 0
        for r, n in enumerate(SIZES):
            sub = regs[off:off + n]
            _ins_sorted(sub, vs[r])
            regs[off:off + n] = sub
            off += n
        return tuple(regs)

    return lax.fori_loop(0, GRPS, grp, init, unroll=2)


def _pop_topk(carry, lane):
    structs, off = [], 0
    for n in SIZES:
        structs.append(list(carry[off:off + n]))
        off += n
    acc = jnp.zeros((L,), jnp.float32)
    for r in range(K):
        h = jnp.maximum(jnp.maximum(structs[0][0], structs[1][0]),
                        jnp.maximum(structs[2][0], structs[3][0]))
        m = _max_splat(h, lane)
        for regs in structs:
            sel = regs[0] == m
            for s in range(len(regs) - 1):
                regs[s] = jnp.where(sel, regs[s + 1], regs[s])
            regs[-1] = jnp.where(sel, NEG, regs[-1])
        acc = jnp.where(lane == r, m, acc)
    return acc


@functools.partial(
    pl.kernel,
    out_type=jax.ShapeDtypeStruct((SC_ROWS, L), jnp.float32),
    mesh=plsc.VectorSubcoreMesh(core_axis_name="c", subcore_axis_name="s"),
    scratch_types=[
        pltpu.VMEM((RCH, B), jnp.float32),
        pltpu.VMEM((RCH, B), jnp.float32),
        pltpu.VMEM((RCH, L), jnp.float32),
        pltpu.VMEM((RCH, L), jnp.float32),
        pltpu.SemaphoreType.DMA,
        pltpu.SemaphoreType.DMA,
        pltpu.SemaphoreType.DMA,
        pltpu.SemaphoreType.DMA,
    ],
)
def _sc_topk(scores_hbm, out_hbm, buf0, buf1, st0, st1,
             isem0, isem1, osem0, osem1):
    cid = lax.axis_index("c")
    sid = lax.axis_index("s")
    wid = sid * NC + cid
    row0 = wid * ROWS_PER_W
    lane = lax.iota(jnp.int32, L)

    def icp(cb, bf, sem):
        return pltpu.make_async_copy(
            scores_hbm.at[pl.ds(row0 + cb * RCH, RCH)], bf, sem)

    def ocp(cb, st, sem):
        return pltpu.make_async_copy(
            st, out_hbm.at[pl.ds(row0 + cb * RCH, RCH)], sem)

    def compute_rows(bf, st, base_row):
        def row(j, _2):
            i_glob = base_row + j
            pv = bf[j, pl.ds((i_glob // L) * L, L)]
            pos = pv.at[jnp.broadcast_to(i_glob % L, (L,))].get(
                mode="promise_in_bounds")
            acc = _pop_topk(_row_candidates(bf, j), lane)
            st[j, :] = jnp.where(lane == K, pos, acc)
            return 0

        lax.fori_loop(0, RCH, row, 0)

    icp(0, buf0, isem0).start()

    def halfpair(h, _):
        cb0 = 2 * h
        icp(cb0 + 1, buf1, isem1).start()
        icp(cb0, buf0, isem0).wait()

        @pl.when(h > 0)
        def _():
            ocp(cb0 - 2, st0, osem0).wait()

        compute_rows(buf0, st0, row0 + cb0 * RCH)
        ocp(cb0, st0, osem0).start()

        @pl.when(h + 1 < NCH // 2)
        def _():
            icp(cb0 + 2, buf0, isem0).start()

        icp(cb0 + 1, buf1, isem1).wait()

        @pl.when(h > 0)
        def _():
            ocp(cb0 - 1, st1, osem1).wait()

        compute_rows(buf1, st1, row0 + (cb0 + 1) * RCH)
        ocp(cb0 + 1, st1, osem1).start()
        return 0

    lax.fori_loop(0, NCH // 2, halfpair, 0)
    ocp(NCH - 2, st0, osem0).wait()
    ocp(NCH - 1, st1, osem1).wait()


def _tc_body(x_ref, o_ref):
    i = pl.program_id(0)
    x = x_ref[...]
    base = SC_ROWS + i * RBLK
    rows = base + lax.broadcasted_iota(jnp.int32, (RBLK, B), 0)
    cols = lax.broadcasted_iota(jnp.int32, (RBLK, B), 1)
    pos = jnp.sum(jnp.where(rows == cols, x, 0.0), axis=1)
    col16 = lax.broadcasted_iota(jnp.int32, (RBLK, L), 1)
    structs = [[jnp.full((RBLK, 128), NEG, jnp.float32) for _ in range(n)]
               for n in SIZES]
    for g in range(B // 512):
        vs = [x[:, (4 * g + i2) * 128:(4 * g + i2 + 1) * 128]
              for i2 in range(4)]
        for a, b in ((0, 1), (2, 3), (0, 2), (1, 3), (1, 2)):
            hi = jnp.maximum(vs[a], vs[b])
            vs[b] = jnp.minimum(vs[a], vs[b])
            vs[a] = hi
        for r in range(4):
            _ins_sorted(structs[r], vs[r])
    out = jnp.zeros((RBLK, L), jnp.float32)
    for r in range(K):
        h = jnp.maximum(jnp.maximum(structs[0][0], structs[1][0]),
                        jnp.maximum(structs[2][0], structs[3][0]))
        m = jnp.max(h, axis=1, keepdims=True)
        for regs in structs:
            sel = regs[0] == m
            for s in range(len(regs) - 1):
                regs[s] = jnp.where(sel, regs[s + 1], regs[s])
            regs[-1] = jnp.where(sel, NEG, regs[-1])
        out = jnp.where(col16 == r, m, out)
    out = jnp.where(col16 == K, pos[:, None], out)
    o_ref[...] = out


def _tc_topk(scores):
    return pl.pallas_call(
        _tc_body,
        grid=(TC_ROWS // RBLK,),
        in_specs=[pl.BlockSpec((RBLK, B), lambda i: (i + SC_ROWS // RBLK, 0))],
        out_specs=pl.BlockSpec((RBLK, L), lambda i: (i, 0)),
        out_shape=jax.ShapeDtypeStruct((TC_ROWS, L), jnp.float32),
    )(scores)


def _finish_body(x_ref, o_ref):
    x = x_ref[...]
    vals = x[:, :K]
    pos = x[:, K:K + 1]
    diff = pos - vals
    sig = 1.0 / (1.0 + jnp.exp(-diff))
    bpr = -jnp.log(jnp.maximum(sig, 1e-8))
    col = lax.broadcasted_iota(jnp.int32, (B, K), 1).astype(jnp.float32)
    w = 1.0 / jnp.log2(col + 2.0)
    g = jnp.sum((vals > pos).astype(jnp.float32), axis=1)
    diag_in = (pos[:, 0] >= vals[:, K - 1]).astype(jnp.float32)
    diag_term = diag_in * (0.6931471805599453 / jnp.log2(g + 2.0))
    row_sum = jnp.sum(w * bpr, axis=1) - diag_term
    total = jnp.sum(row_sum)
    cnt = jnp.float32(K) * B - jnp.sum(diag_in)
    o_ref[0, 0] = total / jnp.maximum(cnt, 1.0)


def _finish(packed):
    return pl.pallas_call(
        _finish_body,
        out_shape=jax.ShapeDtypeStruct((1, 1), jnp.float32),
        out_specs=pl.BlockSpec(memory_space=pltpu.SMEM),
    )(packed)


def kernel(scores):
    packed_tc = _tc_topk(scores)
    packed_sc = _sc_topk(scores)
    loss = _finish(jnp.concatenate([packed_sc, packed_tc], axis=0))
    return loss[0, 0]


# speedup vs baseline: 1.1158x; 1.1158x over previous
import functools

import jax
import jax.numpy as jnp
from jax import lax
from jax.experimental import pallas as pl
from jax.experimental.pallas import tpu as pltpu
from jax.experimental.pallas import tpu_sc as plsc

B = 4096
K = 10
L = 16
NC = 2
NS = 16
NW = NC * NS
SC_ROWS = 1536
TC_ROWS = B - SC_ROWS
RBLK = 256
ROWS_PER_W = SC_ROWS // NW
RCH = 8
NCH = ROWS_PER_W // RCH
CHUNKS = B // L
NEG = -3.0e38


def _max_splat(x, lane):
    for k in (1, 2, 4, 8):
        x = jnp.maximum(x, x.at[lane ^ k].get(mode="promise_in_bounds"))
    return x


SIZES = (K, 5, 3, 2)
GRPS = CHUNKS // 4


def _ins_sorted(regs, v):
    n = len(regs)
    for i in range(n):
        if i < n - 1:
            lo = jnp.minimum(regs[i], v)
            regs[i] = jnp.maximum(regs[i], v)
            v = lo
        else:
            regs[i] = jnp.maximum(regs[i], v)


def _row_candidates(buf, j):
    init = tuple(jnp.full((L,), NEG, jnp.float32) for _ in range(sum(SIZES)))

    def grp(g, carry):
        regs = list(carry)
        vs = [buf[j, pl.ds((4 * g + i) * L, L)] for i in range(4)]
        for a, b in ((0, 1), (2, 3), (0, 2), (1, 3), (1, 2)):
            hi = jnp.maximum(vs[a], vs[b])
            vs[b] = jnp.minimum(vs[a], vs[b])
            vs[a] = hi
        off = 0
        for r, n in enumerate(SIZES):
            sub = regs[off:off + n]
            _ins_sorted(sub, vs[r])
            regs[off:off + n] = sub
            off += n
        return tuple(regs)

    return lax.fori_loop(0, GRPS, grp, init, unroll=2)


def _pop_topk(carry, lane):
    structs, off = [], 0
    for n in SIZES:
        structs.append(list(carry[off:off + n]))
        off += n
    acc = jnp.zeros((L,), jnp.float32)
    for r in range(K):
        h = jnp.maximum(jnp.maximum(structs[0][0], structs[1][0]),
                        jnp.maximum(structs[2][0], structs[3][0]))
        m = _max_splat(h, lane)
        for regs in structs:
            sel = regs[0] == m
            for s in range(len(regs) - 1):
                regs[s] = jnp.where(sel, regs[s + 1], regs[s])
            regs[-1] = jnp.where(sel, NEG, regs[-1])
        acc = jnp.where(lane == r, m, acc)
    return acc


@functools.partial(
    pl.kernel,
    out_type=jax.ShapeDtypeStruct((SC_ROWS, L), jnp.float32),
    mesh=plsc.VectorSubcoreMesh(core_axis_name="c", subcore_axis_name="s"),
    scratch_types=[
        pltpu.VMEM((RCH, B), jnp.float32),
        pltpu.VMEM((RCH, B), jnp.float32),
        pltpu.VMEM((RCH, L), jnp.float32),
        pltpu.VMEM((RCH, L), jnp.float32),
        pltpu.SemaphoreType.DMA,
        pltpu.SemaphoreType.DMA,
        pltpu.SemaphoreType.DMA,
        pltpu.SemaphoreType.DMA,
    ],
)
def _sc_topk(scores_hbm, out_hbm, buf0, buf1, st0, st1,
             isem0, isem1, osem0, osem1):
    cid = lax.axis_index("c")
    sid = lax.axis_index("s")
    wid = sid * NC + cid
    row0 = wid * ROWS_PER_W
    lane = lax.iota(jnp.int32, L)

    def icp(cb, bf, sem):
        return pltpu.make_async_copy(
            scores_hbm.at[pl.ds(row0 + cb * RCH, RCH)], bf, sem)

    def ocp(cb, st, sem):
        return pltpu.make_async_copy(
            st, out_hbm.at[pl.ds(row0 + cb * RCH, RCH)], sem)

    def compute_rows(bf, st, base_row):
        def row(j, _2):
            i_glob = base_row + j
            pv = bf[j, pl.ds((i_glob // L) * L, L)]
            pos = pv.at[jnp.broadcast_to(i_glob % L, (L,))].get(
                mode="promise_in_bounds")
            acc = _pop_topk(_row_candidates(bf, j), lane)
            st[j, :] = jnp.where(lane == K, pos, acc)
            return 0

        lax.fori_loop(0, RCH, row, 0)

    icp(0, buf0, isem0).start()

    def halfpair(h, _):
        cb0 = 2 * h
        icp(cb0 + 1, buf1, isem1).start()
        icp(cb0, buf0, isem0).wait()

        @pl.when(h > 0)
        def _():
            ocp(cb0 - 2, st0, osem0).wait()

        compute_rows(buf0, st0, row0 + cb0 * RCH)
        ocp(cb0, st0, osem0).start()

        @pl.when(h + 1 < NCH // 2)
        def _():
            icp(cb0 + 2, buf0, isem0).start()

        icp(cb0 + 1, buf1, isem1).wait()

        @pl.when(h > 0)
        def _():
            ocp(cb0 - 1, st1, osem1).wait()

        compute_rows(buf1, st1, row0 + (cb0 + 1) * RCH)
        ocp(cb0 + 1, st1, osem1).start()
        return 0

    lax.fori_loop(0, NCH // 2, halfpair, 0)
    ocp(NCH - 2, st0, osem0).wait()
    ocp(NCH - 1, st1, osem1).wait()


def _tc_body(x_ref, o_ref):
    i = pl.program_id(0)
    x = x_ref[...]
    base = SC_ROWS + i * RBLK
    rows = base + lax.broadcasted_iota(jnp.int32, (RBLK, B), 0)
    cols = lax.broadcasted_iota(jnp.int32, (RBLK, B), 1)
    pos = jnp.sum(jnp.where(rows == cols, x, 0.0), axis=1)
    col16 = lax.broadcasted_iota(jnp.int32, (RBLK, L), 1)
    structs = [[jnp.full((RBLK, 128), NEG, jnp.float32) for _ in range(n)]
               for n in SIZES]
    for g in range(B // 512):
        vs = [x[:, (4 * g + i2) * 128:(4 * g + i2 + 1) * 128]
              for i2 in range(4)]
        for a, b in ((0, 1), (2, 3), (0, 2), (1, 3), (1, 2)):
            hi = jnp.maximum(vs[a], vs[b])
            vs[b] = jnp.minimum(vs[a], vs[b])
            vs[a] = hi
        for r in range(4):
            _ins_sorted(structs[r], vs[r])
    out = jnp.zeros((RBLK, L), jnp.float32)
    for r in range(K):
        h = jnp.maximum(jnp.maximum(structs[0][0], structs[1][0]),
                        jnp.maximum(structs[2][0], structs[3][0]))
        m = jnp.max(h, axis=1, keepdims=True)
        for regs in structs:
            sel = regs[0] == m
            for s in range(len(regs) - 1):
                regs[s] = jnp.where(sel, regs[s + 1], regs[s])
            regs[-1] = jnp.where(sel, NEG, regs[-1])
        out = jnp.where(col16 == r, m, out)
    out = jnp.where(col16 == K, pos[:, None], out)
    o_ref[...] = out


def _tc_topk(scores):
    return pl.pallas_call(
        _tc_body,
        grid=(TC_ROWS // RBLK,),
        in_specs=[pl.BlockSpec((RBLK, B), lambda i: (i + SC_ROWS // RBLK, 0))],
        out_specs=pl.BlockSpec((RBLK, L), lambda i: (i, 0)),
        out_shape=jax.ShapeDtypeStruct((TC_ROWS, L), jnp.float32),
    )(scores)


def _finish_body(x_ref, o_ref):
    x = x_ref[...]
    vals = x[:, :K]
    pos = x[:, K:K + 1]
    diff = pos - vals
    sig = 1.0 / (1.0 + jnp.exp(-diff))
    bpr = -jnp.log(jnp.maximum(sig, 1e-8))
    col = lax.broadcasted_iota(jnp.int32, (B, K), 1).astype(jnp.float32)
    w = 1.0 / jnp.log2(col + 2.0)
    g = jnp.sum((vals > pos).astype(jnp.float32), axis=1)
    diag_in = (pos[:, 0] >= vals[:, K - 1]).astype(jnp.float32)
    diag_term = diag_in * (0.6931471805599453 / jnp.log2(g + 2.0))
    row_sum = jnp.sum(w * bpr, axis=1) - diag_term
    total = jnp.sum(row_sum)
    cnt = jnp.float32(K) * B - jnp.sum(diag_in)
    o_ref[0, 0] = total / jnp.maximum(cnt, 1.0)


def _finish(packed):
    return pl.pallas_call(
        _finish_body,
        out_shape=jax.ShapeDtypeStruct((1, 1), jnp.float32),
        out_specs=pl.BlockSpec(memory_space=pltpu.SMEM),
    )(packed)


def kernel(scores):
    packed_tc = _tc_topk(scores)
    packed_sc = _sc_topk(scores)
    loss = _finish(jnp.concatenate([packed_sc, packed_tc], axis=0))
    return loss[0, 0]
